# Initial kernel scaffold; baseline (speedup 1.0000x reference)
import jax
import jax.numpy as jnp
from jax.experimental import pallas as pl


def kernel(x, edge_index, W1, b1, W2, b2):
    raise NotImplementedError("write your pallas kernel here")


# speedup vs baseline: 16.5202x; 16.5202x over previous
import functools

import jax
import jax.numpy as jnp
from jax import lax
from jax.experimental import pallas as pl
from jax.experimental.pallas import tpu as pltpu
from jax.experimental.pallas import tpu_sc as plsc

WIN = 128


def _zero_fill(ref, rows, cols):
    zv = jnp.zeros((16,), jnp.float32)
    cpr = cols // 16

    def body(i, _):
        r = i // cpr
        c = i % cpr
        ref[r, pl.ds(c * 16, 16)] = zv
        return 0

    lax.fori_loop(0, rows * cpr, body, 0)


def _zero_fill_1d(ref, n):
    zv = jnp.zeros((16,), jnp.float32)

    def body(i, _):
        ref[pl.ds(i * 16, 16)] = zv
        return 0

    lax.fori_loop(0, n // 16, body, 0)


@functools.cache
def _deg_kernel(n_nodes, e_pad, nc, ns):
    nw = nc * ns
    per = e_pad // nw
    nwin = per // WIN
    chunk = ((n_nodes + ns * 8 - 1) // (ns * 8)) * 8
    n_out = ns * chunk
    assert n_out >= n_nodes + 8
    zn = ((chunk + 15) // 16) * 16
    mesh = plsc.VectorSubcoreMesh(core_axis_name="c", subcore_axis_name="s")

    @functools.partial(
        pl.kernel,
        out_type=jax.ShapeDtypeStruct((nc * n_out,), jnp.float32),
        mesh=mesh,
        scratch_types=dict(
            acc=pltpu.VMEM_SHARED((n_out,), jnp.float32),
            idx=pltpu.VMEM((WIN,), jnp.int32),
            ones=pltpu.VMEM((WIN,), jnp.float32),
            zbuf=pltpu.VMEM((zn,), jnp.float32),
            sem=pltpu.SemaphoreType.DMA,
        ),
    )
    def k(dst_hbm, out_hbm, acc, idx, ones, zbuf, sem):
        cid = lax.axis_index("c")
        sid = lax.axis_index("s")
        wid = sid * nc + cid

        _zero_fill_1d(zbuf, zn)
        ov = jnp.ones((16,), jnp.float32)
        for i in range(WIN // 16):
            ones[pl.ds(i * 16, 16)] = ov
        pltpu.sync_copy(zbuf.at[pl.ds(0, chunk)], acc.at[pl.ds(sid * chunk, chunk)])
        plsc.subcore_barrier()

        base = wid * per

        def body(g, _):
            pltpu.sync_copy(dst_hbm.at[pl.ds(base + g * WIN, WIN)], idx)
            pltpu.sync_copy(ones, acc.at[idx], add=True)
            return 0

        lax.fori_loop(0, nwin, body, 0)
        plsc.subcore_barrier()
        pltpu.sync_copy(acc.at[pl.ds(sid * chunk, chunk)], zbuf.at[pl.ds(0, chunk)])
        pltpu.sync_copy(
            zbuf.at[pl.ds(0, chunk)],
            out_hbm.at[pl.ds(cid * n_out + sid * chunk, chunk)],
        )

    return k, n_out


@functools.cache
def _agg_kernel(n_nodes, d, e_pad, nc, ns):
    nw = nc * ns
    per = e_pad // nw
    nwin = per // WIN
    zrows = 64
    n_rows = ((n_nodes + 8 + ns * zrows - 1) // (ns * zrows)) * (ns * zrows)
    n_acc = n_rows
    wb = n_rows // ns
    zc = wb // zrows
    mesh = plsc.VectorSubcoreMesh(core_axis_name="c", subcore_axis_name="s")

    @functools.partial(
        pl.kernel,
        out_type=jax.ShapeDtypeStruct((nc, n_rows, d), jnp.float32),
        mesh=mesh,
        scratch_types=dict(
            acc=pltpu.VMEM_SHARED((n_acc, d), jnp.float32),
            idx_s=pltpu.VMEM((WIN,), jnp.int32),
            idx_d=pltpu.VMEM((WIN,), jnp.int32),
            rows=pltpu.VMEM((WIN, d), jnp.float32),
            zbuf=pltpu.VMEM((zrows, d), jnp.float32),
            sem=pltpu.SemaphoreType.DMA,
        ),
    )
    def k(hs_hbm, src_hbm, dst_hbm, out_hbm, acc, idx_s, idx_d, rows, zbuf, sem):
        cid = lax.axis_index("c")
        sid = lax.axis_index("s")
        wid = sid * nc + cid

        _zero_fill(zbuf, zrows, d)
        zbase = sid * zc * zrows

        def zbody(i, _):
            pltpu.sync_copy(zbuf, acc.at[pl.ds(zbase + i * zrows, zrows)])
            return 0

        lax.fori_loop(0, zc, zbody, 0)
        plsc.subcore_barrier()

        base = wid * per

        def body(g, _):
            off = base + g * WIN
            pltpu.sync_copy(src_hbm.at[pl.ds(off, WIN)], idx_s)
            pltpu.sync_copy(dst_hbm.at[pl.ds(off, WIN)], idx_d)
            pltpu.async_copy(hs_hbm.at[idx_s], rows, sem).wait()
            pltpu.sync_copy(rows, acc.at[idx_d], add=True)
            return 0

        lax.fori_loop(0, nwin, body, 0)
        plsc.subcore_barrier()

        def wbody(i, _):
            r = sid * wb + i * zrows
            pltpu.sync_copy(acc.at[pl.ds(r, zrows)], zbuf)
            pltpu.sync_copy(zbuf, out_hbm.at[cid, pl.ds(r, zrows)])
            return 0

        lax.fori_loop(0, zc, wbody, 0)

    return k, n_rows


def _mm_scale_body(x_ref, w_ref, dinv_ref, o_ref):
    h = jnp.dot(x_ref[...], w_ref[...], preferred_element_type=jnp.float32)
    o_ref[...] = h * dinv_ref[...]


def _combine_mm_body(p_ref, hs_ref, dinv_ref, w_ref, b_ref, o_ref, *, n):
    agg = p_ref[0, pl.ds(0, n), :] + p_ref[1, pl.ds(0, n), :]
    z = jnp.maximum((agg + hs_ref[...]) * dinv_ref[...] + b_ref[...], 0.0)
    h = jnp.dot(z, w_ref[...], preferred_element_type=jnp.float32)
    o_ref[...] = h * dinv_ref[...]


def _combine_final_body(p_ref, hs_ref, dinv_ref, b_ref, o_ref, *, n):
    agg = p_ref[0, pl.ds(0, n), :] + p_ref[1, pl.ds(0, n), :]
    o_ref[...] = jnp.maximum((agg + hs_ref[...]) * dinv_ref[...] + b_ref[...], 0.0)


def kernel(x, edge_index, W1, b1, W2, b2):
    n, d_in = x.shape
    d_hid = W1.shape[1]
    e = edge_index.shape[1]
    nc, ns = 2, 16
    nw = nc * ns

    src = edge_index[0].astype(jnp.int32)
    dst = edge_index[1].astype(jnp.int32)

    per = ((e + nw * WIN - 1) // (nw * WIN)) * WIN
    e_pad = per * nw
    npad = e_pad - e
    if npad:
        ar = jnp.arange(npad, dtype=jnp.int32)
        src = jnp.concatenate([src, ar % n])
        dst = jnp.concatenate([dst, n + (ar % 8)])

    kdeg, n_out = _deg_kernel(n, e_pad, nc, ns)
    degp = kdeg(dst)
    deg = degp[:n] + degp[n_out : n_out + n] + 1.0
    dinv = (deg ** -0.5).reshape(n, 1)

    mm_scale = pl.pallas_call(
        _mm_scale_body,
        out_shape=jax.ShapeDtypeStruct((n, d_hid), jnp.float32),
    )
    combine_mm = pl.pallas_call(
        functools.partial(_combine_mm_body, n=n),
        out_shape=jax.ShapeDtypeStruct((n, d_hid), jnp.float32),
    )
    combine_final = pl.pallas_call(
        functools.partial(_combine_final_body, n=n),
        out_shape=jax.ShapeDtypeStruct((n, d_hid), jnp.float32),
    )
    agg, _ = _agg_kernel(n, d_hid, e_pad, nc, ns)

    b1r = b1.reshape(1, d_hid)
    b2r = b2.reshape(1, d_hid)

    hs1 = mm_scale(x, W1, dinv)
    p1 = agg(hs1, src, dst)
    hs2 = combine_mm(p1, hs1, dinv, W2, b1r)
    p2 = agg(hs2, src, dst)
    out = combine_final(p2, hs2, dinv, b2r)
    return out


# speedup vs baseline: 29.6654x; 1.7957x over previous
import functools

import jax
import jax.numpy as jnp
from jax import lax
from jax.experimental import pallas as pl
from jax.experimental.pallas import tpu as pltpu
from jax.experimental.pallas import tpu_sc as plsc

WIN = 128


def _zero_fill(ref, rows, cols):
    zv = jnp.zeros((16,), jnp.float32)
    cpr = cols // 16

    def body(i, _):
        ref[i // cpr, pl.ds((i % cpr) * 16, 16)] = zv
        return 0

    lax.fori_loop(0, rows * cpr, body, 0)


@functools.cache
def _deg_kernel(n_nodes, nwin, nc, ns):
    chunk = ((n_nodes + ns * 8 - 1) // (ns * 8)) * 8
    n_out = ns * chunk
    assert n_out >= n_nodes + 8
    zn = ((chunk + 15) // 16) * 16
    mesh = plsc.VectorSubcoreMesh(core_axis_name="c", subcore_axis_name="s")

    @functools.partial(
        pl.kernel,
        out_type=jax.ShapeDtypeStruct((nc * n_out,), jnp.float32),
        mesh=mesh,
        scratch_types=dict(
            acc=pltpu.VMEM_SHARED((n_out,), jnp.float32),
            idx=pltpu.VMEM((nwin, WIN), jnp.int32),
            ones=pltpu.VMEM((WIN,), jnp.float32),
            zbuf=pltpu.VMEM((zn,), jnp.float32),
            sem=pltpu.SemaphoreType.DMA,
            ssem=pltpu.SemaphoreType.DMA,
        ),
    )
    def k(dst_hbm, out_hbm, acc, idx, ones, zbuf, sem, ssem):
        cid = lax.axis_index("c")
        sid = lax.axis_index("s")
        wid = sid * nc + cid

        zv = jnp.zeros((16,), jnp.float32)
        ov = jnp.ones((16,), jnp.float32)

        def zfill(i, _):
            zbuf[pl.ds(i * 16, 16)] = zv
            return 0

        lax.fori_loop(0, zn // 16, zfill, 0)
        for i in range(WIN // 16):
            ones[pl.ds(i * 16, 16)] = ov
        pltpu.sync_copy(zbuf.at[pl.ds(0, chunk)], acc.at[pl.ds(sid * chunk, chunk)])
        pltpu.async_copy(dst_hbm.at[wid], idx, sem)
        plsc.subcore_barrier()
        pltpu.make_async_copy(dst_hbm.at[wid], idx, sem).wait()

        def body(g, _):
            pltpu.make_async_copy(ones, acc.at[idx.at[g]], ssem).start(add=True)
            return 0

        lax.fori_loop(0, nwin, body, 0)

        def drain(g, _):
            pltpu.make_async_copy(ones, acc.at[idx.at[g]], ssem).wait()
            return 0

        lax.fori_loop(0, nwin, drain, 0)
        plsc.subcore_barrier()
        pltpu.sync_copy(acc.at[pl.ds(sid * chunk, chunk)], zbuf.at[pl.ds(0, chunk)])
        pltpu.sync_copy(
            zbuf.at[pl.ds(0, chunk)],
            out_hbm.at[pl.ds(cid * n_out + sid * chunk, chunk)],
        )

    return k, n_out


@functools.cache
def _agg_kernel(n_nodes, d, nwin, nc, ns):
    assert nwin % 4 == 0
    zrows = 64
    n_rows = ((n_nodes + 8 + ns * zrows - 1) // (ns * zrows)) * (ns * zrows)
    wb = n_rows // ns
    zc = wb // zrows
    mesh = plsc.VectorSubcoreMesh(core_axis_name="c", subcore_axis_name="s")

    @functools.partial(
        pl.kernel,
        out_type=jax.ShapeDtypeStruct((nc, n_rows, d), jnp.float32),
        mesh=mesh,
        scratch_types=dict(
            acc=pltpu.VMEM_SHARED((n_rows, d), jnp.float32),
            idx=pltpu.VMEM((4, 2, WIN), jnp.int32),
            rows=pltpu.VMEM((2, WIN, d), jnp.float32),
            zbuf=pltpu.VMEM((zrows, d), jnp.float32),
            is0=pltpu.SemaphoreType.DMA,
            is1=pltpu.SemaphoreType.DMA,
            is2=pltpu.SemaphoreType.DMA,
            is3=pltpu.SemaphoreType.DMA,
            gs0=pltpu.SemaphoreType.DMA,
            gs1=pltpu.SemaphoreType.DMA,
        ),
    )
    def k(hs_hbm, edges_hbm, out_hbm, acc, idx, rows, zbuf,
          is0, is1, is2, is3, gs0, gs1):
        isem = (is0, is1, is2, is3)
        gsem = (gs0, gs1)
        cid = lax.axis_index("c")
        sid = lax.axis_index("s")
        wid = sid * nc + cid

        def idxload(g, r):
            return pltpu.make_async_copy(
                edges_hbm.at[wid, lax.rem(g, nwin)], idx.at[r], isem[r]
            )

        def gather(r, b):
            return pltpu.make_async_copy(
                hs_hbm.at[idx.at[r, 0]], rows.at[b], gsem[b]
            )

        idxload(0, 0).start()
        idxload(1, 1).start()

        _zero_fill(zbuf, zrows, d)
        zbase = sid * wb

        def zbody(i, _):
            pltpu.sync_copy(zbuf, acc.at[pl.ds(zbase + i * zrows, zrows)])
            return 0

        lax.fori_loop(0, zc, zbody, 0)
        idxload(0, 0).wait()
        gather(0, 0).start()
        plsc.subcore_barrier()

        def quad(i, _):
            g0 = i * 4
            for u in range(4):
                g = g0 + u
                b, bn = u % 2, (u + 1) % 2
                r, rn, rp = u, (u + 1) % 4, (u + 2) % 4
                gather(r, b).wait()
                idxload(g + 1, rn).wait()
                gather(rn, bn).start()
                pltpu.sync_copy(rows.at[b], acc.at[idx.at[r, 1]], add=True)
                idxload(g + 2, rp).start()
            return 0

        lax.fori_loop(0, nwin // 4, quad, 0)
        gather(0, 0).wait()
        idxload(1, 1).wait()
        plsc.subcore_barrier()

        def wbody(i, _):
            rr = sid * wb + i * zrows
            pltpu.sync_copy(acc.at[pl.ds(rr, zrows)], zbuf)
            pltpu.sync_copy(zbuf, out_hbm.at[cid, pl.ds(rr, zrows)])
            return 0

        lax.fori_loop(0, zc, wbody, 0)

    return k, n_rows


def _mm_body(x_ref, w_ref, o_ref):
    o_ref[...] = jnp.dot(x_ref[...], w_ref[...], preferred_element_type=jnp.float32)


def _scale_body(h_ref, dinv_ref, o_ref):
    o_ref[...] = h_ref[...] * dinv_ref[...]


def _combine_mm_body(p_ref, hs_ref, dinv_ref, w_ref, b_ref, o_ref, *, n):
    agg = p_ref[0, pl.ds(0, n), :] + p_ref[1, pl.ds(0, n), :]
    z = jnp.maximum((agg + hs_ref[...]) * dinv_ref[...] + b_ref[...], 0.0)
    h = jnp.dot(z, w_ref[...], preferred_element_type=jnp.float32)
    o_ref[...] = h * dinv_ref[...]


def _combine_final_body(p_ref, hs_ref, dinv_ref, b_ref, o_ref, *, n):
    agg = p_ref[0, pl.ds(0, n), :] + p_ref[1, pl.ds(0, n), :]
    o_ref[...] = jnp.maximum((agg + hs_ref[...]) * dinv_ref[...] + b_ref[...], 0.0)


def kernel(x, edge_index, W1, b1, W2, b2):
    n, d_in = x.shape
    d_hid = W1.shape[1]
    e = edge_index.shape[1]
    nc, ns = 2, 16
    nw = nc * ns

    src = edge_index[0].astype(jnp.int32)
    dst = edge_index[1].astype(jnp.int32)

    nwin = ((e + nw * WIN - 1) // (nw * WIN) + 7) // 8 * 8
    e_pad = nwin * WIN * nw
    npad = e_pad - e
    if npad:
        ar = jnp.arange(npad, dtype=jnp.int32)
        src = jnp.concatenate([src, ar % n])
        dst = jnp.concatenate([dst, n + (ar % 8)])
    src3 = src.reshape(nw, nwin, WIN)
    dst3 = dst.reshape(nw, nwin, WIN)
    edges4 = jnp.stack([src3, dst3], axis=2)

    mm = pl.pallas_call(
        _mm_body, out_shape=jax.ShapeDtypeStruct((n, d_hid), jnp.float32)
    )
    scale = pl.pallas_call(
        _scale_body, out_shape=jax.ShapeDtypeStruct((n, d_hid), jnp.float32)
    )
    combine_mm = pl.pallas_call(
        functools.partial(_combine_mm_body, n=n),
        out_shape=jax.ShapeDtypeStruct((n, d_hid), jnp.float32),
    )
    combine_final = pl.pallas_call(
        functools.partial(_combine_final_body, n=n),
        out_shape=jax.ShapeDtypeStruct((n, d_hid), jnp.float32),
    )
    kdeg, n_out = _deg_kernel(n, nwin, nc, ns)
    agg, _ = _agg_kernel(n, d_hid, nwin, nc, ns)

    degp = kdeg(dst3)
    h1 = mm(x, W1)
    deg = degp[:n] + degp[n_out : n_out + n] + 1.0
    dinv = (deg ** -0.5).reshape(n, 1)

    b1r = b1.reshape(1, d_hid)
    b2r = b2.reshape(1, d_hid)

    hs1 = scale(h1, dinv)
    p1 = agg(hs1, edges4)
    hs2 = combine_mm(p1, hs1, dinv, W2, b1r)
    p2 = agg(hs2, edges4)
    out = combine_final(p2, hs2, dinv, b2r)
    return out
